# baseline (speedup 1.0000x reference)
import functools

import jax
import jax.numpy as jnp
from jax import lax
from jax.experimental import pallas as pl
from jax.experimental.pallas import tpu as pltpu
from jax.experimental.pallas import tpu_sc as plsc

NC = 2
NS = 16
NW = NC * NS
SUB = 80
K = 5
CH = SUB * K


@functools.partial(jax.jit, static_argnums=(2, 3))
def _sc_gather(data, indices_3d, b, d):
    idx = indices_3d.astype(jnp.int32).reshape(b)
    b_per_w = b // NW
    nsteps = b_per_w // CH
    mesh = plsc.VectorSubcoreMesh(
        core_axis_name="c", subcore_axis_name="s",
        num_cores=NC, num_subcores=NS)

    @functools.partial(
        pl.kernel,
        out_type=jax.ShapeDtypeStruct((b, 2 * d), jnp.float32),
        mesh=mesh,
        scratch_types=[
            pltpu.VMEM((b_per_w,), jnp.int32),
            pltpu.VMEM((2, CH, 2 * d), jnp.float32),
            pltpu.SemaphoreType.DMA,
            pltpu.SemaphoreType.DMA,
            pltpu.SemaphoreType.DMA,
            pltpu.SemaphoreType.DMA,
        ],
        compiler_params=pltpu.CompilerParams(use_tc_tiling_on_sc=False),
    )
    def body(data_hbm, idx_hbm, out_hbm, idx_v, rows_v,
             gsem0, gsem1, wsem0, wsem1):
        gsems = (gsem0, gsem1)
        wsems = (wsem0, wsem1)
        wid = lax.axis_index("s") * NC + lax.axis_index("c")
        pltpu.sync_copy(idx_hbm.at[pl.ds(wid * b_per_w, b_per_w)], idx_v)

        def g_descs(c, buf):
            return [
                pltpu.make_async_copy(
                    data_hbm.at[idx_v.at[pl.ds(c * CH + j * SUB, SUB)]],
                    rows_v.at[buf, pl.ds(j * SUB, SUB)],
                    gsems[buf])
                for j in range(K)
            ]

        def g_start(c, buf):
            for dd in g_descs(c, buf):
                dd.start()

        def g_wait(c, buf):
            for dd in g_descs(c, buf):
                dd.wait()

        def w_desc(c, buf):
            return pltpu.make_async_copy(
                rows_v.at[buf, :, pl.ds(0, d)],
                out_hbm.at[pl.ds(wid * b_per_w + c * CH, CH), pl.ds(0, d)],
                wsems[buf])

        assert nsteps >= 4 and nsteps % 2 == 0
        g_start(0, 0)
        g_start(1, 1)
        g_wait(0, 0)
        w_desc(0, 0).start()

        @pl.loop(0, (nsteps - 2) // 2)
        def _(g):
            for db in range(2):
                c = 2 * g + 1 + db
                buf = (1 + db) % 2
                nbuf = 1 - buf
                w_desc(c - 1, nbuf).wait()
                g_start(c + 1, nbuf)
                g_wait(c, buf)
                w_desc(c, buf).start()

        c_last = nsteps - 1
        g_wait(c_last, c_last % 2)
        w_desc(c_last, c_last % 2).start()
        w_desc(c_last - 1, (c_last - 1) % 2).wait()
        w_desc(c_last, c_last % 2).wait()

    return body(data, idx)


def kernel(data, indices):
    orig_shape = indices.shape
    m = orig_shape[-1]
    assert m == 1 and data.ndim == 2
    d = data.shape[1]
    b = indices.size
    nsteps = b // (NW * CH)
    assert b % (NW * CH) == 0 and nsteps >= 4 and nsteps % 2 == 0
    data_p = jnp.pad(data, ((0, 0), (0, d)))
    out = _sc_gather(data_p, indices, b, d)
    return out[:, :d].reshape(orig_shape[:-1] + (d,))


# speedup vs baseline: 1.0034x; 1.0034x over previous
import functools

import jax
import jax.numpy as jnp
from jax import lax
from jax.experimental import pallas as pl
from jax.experimental.pallas import tpu as pltpu
from jax.experimental.pallas import tpu_sc as plsc

NC = 2
NS = 16
NW = NC * NS
SUB = 128
K = 4
CH = SUB * K


@functools.partial(jax.jit, static_argnums=(2, 3))
def _sc_gather(data, indices_3d, b, d):
    idx = indices_3d.astype(jnp.int32).reshape(b)
    b_per_w = b // NW
    nsteps = b_per_w // CH
    mesh = plsc.VectorSubcoreMesh(
        core_axis_name="c", subcore_axis_name="s",
        num_cores=NC, num_subcores=NS)

    @functools.partial(
        pl.kernel,
        out_type=jax.ShapeDtypeStruct((b, 2 * d), jnp.float32),
        mesh=mesh,
        scratch_types=[
            pltpu.VMEM((b_per_w,), jnp.int32),
            pltpu.VMEM((3, CH, d), jnp.float32),
            pltpu.SemaphoreType.DMA,
            pltpu.SemaphoreType.DMA,
            pltpu.SemaphoreType.DMA,
            pltpu.SemaphoreType.DMA,
            pltpu.SemaphoreType.DMA,
            pltpu.SemaphoreType.DMA,
        ],
        compiler_params=pltpu.CompilerParams(use_tc_tiling_on_sc=False),
    )
    def body(data_hbm, idx_hbm, out_hbm, idx_v, rows_v,
             gsem0, gsem1, gsem2, wsem0, wsem1, wsem2):
        gsems = (gsem0, gsem1, gsem2)
        wsems = (wsem0, wsem1, wsem2)
        wid = lax.axis_index("s") * NC + lax.axis_index("c")
        pltpu.sync_copy(idx_hbm.at[pl.ds(wid * b_per_w, b_per_w)], idx_v)

        def g_descs(c, buf):
            return [
                pltpu.make_async_copy(
                    data_hbm.at[idx_v.at[pl.ds(c * CH + j * SUB, SUB)]],
                    rows_v.at[buf, pl.ds(j * SUB, SUB)],
                    gsems[buf])
                for j in range(K)
            ]

        def g_start(c, buf):
            for dd in g_descs(c, buf):
                dd.start()

        def g_wait(c, buf):
            for dd in g_descs(c, buf):
                dd.wait()

        def w_desc(c, buf):
            return pltpu.make_async_copy(
                rows_v.at[buf],
                out_hbm.at[pl.ds(wid * b_per_w + c * CH, CH), pl.ds(0, d)],
                wsems[buf])

        assert nsteps >= 6 and (nsteps - 5) % 3 == 0
        g_start(0, 0)
        g_start(1, 1)
        g_start(2, 2)
        for c in (0, 1):
            g_wait(c, c)
            w_desc(c, c).start()

        @pl.loop(0, (nsteps - 5) // 3)
        def _(g):
            for db in range(3):
                c = 3 * g + 2 + db
                buf = (2 + db) % 3
                nbuf = (buf + 1) % 3
                w_desc(c - 2, nbuf).wait()
                g_start(c + 1, nbuf)
                g_wait(c, buf)
                w_desc(c, buf).start()

        for c in (nsteps - 3, nsteps - 2):
            buf = c % 3
            nbuf = (buf + 1) % 3
            w_desc(c - 2, nbuf).wait()
            g_start(c + 1, nbuf)
            g_wait(c, buf)
            w_desc(c, buf).start()
        c_last = nsteps - 1
        g_wait(c_last, c_last % 3)
        w_desc(c_last, c_last % 3).start()
        for c in (nsteps - 3, nsteps - 2, nsteps - 1):
            w_desc(c, c % 3).wait()

    return body(data, idx)


def kernel(data, indices):
    orig_shape = indices.shape
    m = orig_shape[-1]
    assert m == 1 and data.ndim == 2
    d = data.shape[1]
    b = indices.size
    nsteps = b // (NW * CH)
    assert b % (NW * CH) == 0 and nsteps >= 6 and (nsteps - 5) % 3 == 0
    out = _sc_gather(data, indices, b, d)
    return out[:, :d].reshape(orig_shape[:-1] + (d,))
